# baseline (speedup 1.0000x reference)
import functools

import jax
import jax.numpy as jnp
from jax import lax
from jax.experimental import pallas as pl
from jax.experimental.pallas import tpu as pltpu
from jax.experimental.pallas import tpu_sc as plsc

_E = 8
_H = 768
_HC = _H // 16
_BT = 4096
_N_SC = 8192
_NW = 32


def _top2_tc(logits):
    lt = logits.T
    sub = lax.broadcasted_iota(jnp.int32, lt.shape, 0)
    m1 = jnp.max(lt, axis=0, keepdims=True)
    i1 = jnp.min(jnp.where(lt == m1, sub, _E), axis=0, keepdims=True)
    masked = jnp.where(sub == i1, -jnp.inf, lt)
    m2 = jnp.max(masked, axis=0, keepdims=True)
    i2 = jnp.min(jnp.where(masked == m2, sub, _E), axis=0, keepdims=True)
    e = jnp.exp(m2 - m1)
    den = 1.0 + e
    return (jnp.concatenate([1.0 / den, e / den], axis=0),
            jnp.concatenate([i1, i2], axis=0))


def _router_block(x_ref, w_ref, logits_ref, prob_ref, idx_ref):
    logits = lax.dot_general(
        x_ref[...], w_ref[...], (((1,), (1,)), ((), ())),
        preferred_element_type=jnp.float32,
    )
    logits_ref[...] = logits
    prob, idx = _top2_tc(logits)
    prob_ref[...] = prob
    idx_ref[...] = idx


def _tc_router(hidden_states, weight, n_tc):
    hidden = hidden_states.shape[1]
    return pl.pallas_call(
        _router_block,
        grid=(n_tc // _BT,),
        in_specs=[
            pl.BlockSpec((_BT, hidden), lambda i: (i, 0)),
            pl.BlockSpec((_E, hidden), lambda i: (0, 0)),
        ],
        out_specs=[
            pl.BlockSpec((_BT, _E), lambda i: (i, 0)),
            pl.BlockSpec((2, _BT), lambda i: (0, i)),
            pl.BlockSpec((2, _BT), lambda i: (0, i)),
        ],
        out_shape=[
            jax.ShapeDtypeStruct((n_tc, _E), jnp.float32),
            jax.ShapeDtypeStruct((2, n_tc), jnp.float32),
            jax.ShapeDtypeStruct((2, n_tc), jnp.int32),
        ],
    )(hidden_states, weight)


def _make_sc_router(sc_base, n_sc):
    n_w = n_sc // _NW
    n_g = n_w // 16
    mesh = plsc.VectorSubcoreMesh(core_axis_name="c", subcore_axis_name="s")

    @functools.partial(
        pl.kernel,
        mesh=mesh,
        compiler_params=pltpu.CompilerParams(
            needs_layout_passes=False, use_tc_tiling_on_sc=False),
        out_type=[
            jax.ShapeDtypeStruct((_E, n_sc), jnp.float32),
            jax.ShapeDtypeStruct((2, n_sc), jnp.float32),
            jax.ShapeDtypeStruct((2, n_sc), jnp.int32),
        ],
        scratch_types=[
            pltpu.VMEM((_E, _H), jnp.float32),
            pltpu.VMEM((16, _H), jnp.float32),
            pltpu.VMEM((_E * 16 * 16,), jnp.float32),
            pltpu.VMEM((_E, n_w), jnp.float32),
            pltpu.VMEM((2, n_w), jnp.float32),
            pltpu.VMEM((2, n_w), jnp.int32),
        ],
    )
    def sc_router(x_hbm, w_hbm, lt_hbm, p_hbm, i_hbm,
                  w_v, x_v, st_v, lt_v, p_v, i_v):
        wid = lax.axis_index("s") * 2 + lax.axis_index("c")
        off = wid * n_w
        pltpu.sync_copy(w_hbm, w_v)
        lanes = lax.iota(jnp.int32, 16)
        lanes16 = lanes * 16

        def _rnd(v):
            u = plsc.bitcast(v, jnp.uint32)
            lsb = (u >> jnp.uint32(16)) & jnp.uint32(1)
            r = (u + jnp.uint32(0x7FFF) + lsb) & jnp.uint32(0xFFFF0000)
            return plsc.bitcast(r, jnp.float32)

        def round_w(c, carry):
            base = c * 16
            for e in range(_E):
                w_v[e, pl.ds(base, 16)] = _rnd(w_v[e, pl.ds(base, 16)])
            return carry

        lax.fori_loop(0, _HC, round_w, 0)

        def group(g, carry):
            pltpu.sync_copy(x_hbm.at[pl.ds(sc_base + off + g * 16, 16), :], x_v)
            for q in range(4):
                def chunk(c, accs):
                    base = c * 16
                    xs = [_rnd(x_v[q * 4 + j, pl.ds(base, 16)])
                          for j in range(4)]
                    ws = [w_v[e, pl.ds(base, 16)] for e in range(_E)]
                    return tuple(accs[j * _E + e] + xs[j] * ws[e]
                                 for j in range(4) for e in range(_E))

                accs = lax.fori_loop(
                    0, _HC, chunk,
                    tuple(jnp.zeros((16,), jnp.float32) for _ in range(4 * _E)))
                for j in range(4):
                    t = q * 4 + j
                    for e in range(_E):
                        plsc.store_scatter(
                            st_v, [lanes16 + (e * 256 + t)], accs[j * _E + e])
            rows = []
            for e in range(_E):
                cols = [st_v[pl.ds(e * 256 + c * 16, 16)]
                        for c in range(16)]
                s0 = (((cols[0] + cols[1]) + (cols[2] + cols[3]))
                      + ((cols[4] + cols[5]) + (cols[6] + cols[7])))
                s1 = (((cols[8] + cols[9]) + (cols[10] + cols[11]))
                      + ((cols[12] + cols[13]) + (cols[14] + cols[15])))
                rows.append(s0 + s1)
            m1 = rows[0]
            i1 = jnp.zeros((16,), jnp.int32)
            for e in range(1, _E):
                gt = rows[e] > m1
                m1 = jnp.where(gt, rows[e], m1)
                i1 = jnp.where(gt, e, i1)
            m2 = jnp.full((16,), -jnp.inf, jnp.float32)
            i2 = jnp.zeros((16,), jnp.int32)
            for e in range(_E):
                sel = (rows[e] > m2) & (i1 != e)
                m2 = jnp.where(sel, rows[e], m2)
                i2 = jnp.where(sel, e, i2)
            d = jnp.exp(m2 - m1)
            den = 1.0 + d
            r = 1.0 / den
            r = r * (2.0 - den * r)
            sl = pl.ds(g * 16, 16)
            for e in range(_E):
                lt_v[e, sl] = rows[e]
            p_v[0, sl] = r
            p_v[1, sl] = d * r
            i_v[0, sl] = i1
            i_v[1, sl] = i2
            return carry

        lax.fori_loop(0, n_g, group, 0)
        osl = pl.ds(off, n_w)
        pltpu.sync_copy(lt_v, lt_hbm.at[:, osl])
        pltpu.sync_copy(p_v, p_hbm.at[:, osl])
        pltpu.sync_copy(i_v, i_hbm.at[:, osl])

    return sc_router


def kernel(hidden_states, weight):
    n_tokens = hidden_states.shape[0]
    n_tc = n_tokens - _N_SC
    lt_sc, p_sc, i_sc = _make_sc_router(n_tc, _N_SC)(hidden_states, weight)
    logits_tc, prob_t_tc, idx_t_tc = _tc_router(hidden_states, weight, n_tc)
    logits = jnp.concatenate([logits_tc, lt_sc.T], axis=0)
    prob_t = jnp.concatenate([prob_t_tc, p_sc], axis=1)
    idx_t = jnp.concatenate([idx_t_tc, i_sc], axis=1)
    return (logits, prob_t.T, idx_t.T)


# speedup vs baseline: 4.1165x; 4.1165x over previous
import jax
import jax.numpy as jnp
from jax import lax
from jax.experimental import pallas as pl
from jax.experimental.pallas import tpu as pltpu

_E = 8
_H = 768
_BT = 2048
_NBUF = 4


def _top2(logits):
    lt = logits.T
    sub = lax.broadcasted_iota(jnp.int32, lt.shape, 0)
    m1 = jnp.max(lt, axis=0, keepdims=True)
    i1 = jnp.min(jnp.where(lt == m1, sub, _E), axis=0, keepdims=True)
    masked = jnp.where(sub == i1, -jnp.inf, lt)
    m2 = jnp.max(masked, axis=0, keepdims=True)
    i2 = jnp.min(jnp.where(masked == m2, sub, _E), axis=0, keepdims=True)
    e = jnp.exp(m2 - m1)
    den = 1.0 + e
    return (jnp.concatenate([1.0 / den, e / den], axis=0),
            jnp.concatenate([i1, i2], axis=0))


def _router(n_tokens, x_hbm, w_ref, logits_ref, prob_ref, idx_ref, buf, sems):
    nblk = n_tokens // _BT

    def start(b):
        slot = b % _NBUF
        pltpu.make_async_copy(
            x_hbm.at[pl.ds(b * _BT, _BT), :], buf.at[slot], sems.at[slot]
        ).start()

    for b in range(_NBUF - 1):
        start(b)
    w = w_ref[...]
    for b in range(nblk):
        slot = b % _NBUF
        pltpu.make_async_copy(
            x_hbm.at[pl.ds(b * _BT, _BT), :], buf.at[slot], sems.at[slot]
        ).wait()
        if b + _NBUF - 1 < nblk:
            start(b + _NBUF - 1)
        logits = lax.dot_general(
            buf[slot], w, (((1,), (1,)), ((), ())),
            preferred_element_type=jnp.float32,
        )
        logits_ref[pl.ds(b * _BT, _BT), :] = logits
        prob, idx = _top2(logits)
        prob_ref[:, pl.ds(b * _BT, _BT)] = prob
        idx_ref[:, pl.ds(b * _BT, _BT)] = idx


def kernel(hidden_states, weight):
    n_tokens, hidden = hidden_states.shape
    import functools
    logits, prob_t, idx_t = pl.pallas_call(
        functools.partial(_router, n_tokens),
        in_specs=[
            pl.BlockSpec(memory_space=pl.ANY),
            pl.BlockSpec(memory_space=pltpu.VMEM),
        ],
        out_specs=[
            pl.BlockSpec(memory_space=pltpu.VMEM),
            pl.BlockSpec(memory_space=pltpu.VMEM),
            pl.BlockSpec(memory_space=pltpu.VMEM),
        ],
        out_shape=[
            jax.ShapeDtypeStruct((n_tokens, _E), jnp.float32),
            jax.ShapeDtypeStruct((2, n_tokens), jnp.float32),
            jax.ShapeDtypeStruct((2, n_tokens), jnp.int32),
        ],
        scratch_shapes=[
            pltpu.VMEM((_NBUF, _BT, _H), jnp.float32),
            pltpu.SemaphoreType.DMA((_NBUF,)),
        ],
    )(hidden_states, weight)
    return (logits, prob_t.T, idx_t.T)
